# Initial kernel scaffold; baseline (speedup 1.0000x reference)
import jax
import jax.numpy as jnp
from jax.experimental import pallas as pl


def kernel(Z, g_wih, g_whh, g_bih, g_bhh, g_wl, g_bl, s_wih, s_whh, s_bih, s_bhh, s_wl, s_bl, d_wih, d_whh, d_bih, d_bhh, d_wl, d_bl):
    raise NotImplementedError("write your pallas kernel here")


# speedup vs baseline: 1.1291x; 1.1291x over previous
import jax
import jax.numpy as jnp
from jax.experimental import pallas as pl
from jax.experimental.pallas import tpu as pltpu


def _tg_kernel(
    z_ref,
    g_wih, g_whh, g_bih, g_bhh, g_wl, g_bl,
    s_wih, s_whh, s_bih, s_bhh, s_wl, s_bl,
    d_wih, d_whh, d_bih, d_bhh, wstack,
    out_ref,
    gi_ref, h_ref, hcat_ref,
):
    T, Bb, Dz = z_ref.shape
    H = g_whh.shape[0]
    G3 = 3 * H
    TB = T * Bb
    f32 = jnp.float32

    def run_gru(whh_ref, bhh_ref, store_h):
        whh = whh_ref[...]
        bhh = bhh_ref[...]
        h = jnp.zeros((Bb, H), f32)
        for t in range(T):
            gi = gi_ref[t]
            gh = jnp.dot(h, whh, preferred_element_type=f32) + bhh
            r = jax.nn.sigmoid(gi[:, :H] + gh[:, :H])
            zg = jax.nn.sigmoid(gi[:, H:2 * H] + gh[:, H:2 * H])
            n = jnp.tanh(gi[:, 2 * H:] + r * gh[:, 2 * H:])
            h = (1.0 - zg) * n + zg * h
            store_h(t, h)

    def store_tm(t, h):
        h_ref[t] = h

    def store_cat(t, h):
        hcat_ref[:, t * H:(t + 1) * H] = h

    z_flat = z_ref[...].reshape(TB, Dz)
    gi_ref[...] = (
        jnp.dot(z_flat, g_wih[...], preferred_element_type=f32) + g_bih[...]
    ).reshape(T, Bb, G3)

    run_gru(g_whh, g_bhh, store_tm)

    h1 = h_ref[...].reshape(TB, H)
    e_hat = jax.nn.sigmoid(
        jnp.dot(h1, g_wl[...], preferred_element_type=f32) + g_bl[...])
    gi_ref[...] = (
        jnp.dot(e_hat, s_wih[...], preferred_element_type=f32) + s_bih[...]
    ).reshape(T, Bb, G3)

    run_gru(s_whh, s_bhh, store_tm)

    h2 = h_ref[...].reshape(TB, H)
    h_hat = jax.nn.sigmoid(
        jnp.dot(h2, s_wl[...], preferred_element_type=f32) + s_bl[...])
    gi_ref[...] = (
        jnp.dot(h_hat, d_wih[...], preferred_element_type=f32) + d_bih[...]
    ).reshape(T, Bb, G3)

    run_gru(d_whh, d_bhh, store_cat)

    out_ref[...] = jnp.dot(
        hcat_ref[...], wstack[...], preferred_element_type=f32)


def _block_b(B):
    for c in (256, 128, 64, 32, 16, 8):
        if B % c == 0:
            return c
    return B


def kernel(Z, g_wih, g_whh, g_bih, g_bhh, g_wl, g_bl,
           s_wih, s_whh, s_bih, s_bhh, s_wl, s_bl,
           d_wih, d_whh, d_bih, d_bhh, d_wl, d_bl):
    B, T, Dz = Z.shape
    H = g_whh.shape[0]
    Bb = _block_b(B)
    f32 = jnp.float32

    z_tm = jnp.transpose(Z, (1, 0, 2)).astype(f32)

    eye = jnp.eye(T, dtype=f32)
    wstack = (eye[:, None, :] * d_wl[None, :, 0, None]).reshape(T * H, T)
    wstack = jnp.pad(wstack, ((0, 0), (0, 128 - T)))

    weights = [g_wih, g_whh, g_bih, g_bhh, g_wl, g_bl,
               s_wih, s_whh, s_bih, s_bhh, s_wl, s_bl,
               d_wih, d_whh, d_bih, d_bhh, wstack]
    weights = [w.astype(f32) for w in weights]

    def wspec(w):
        return pl.BlockSpec(w.shape, lambda i, nd=w.ndim: (0,) * nd)

    out = pl.pallas_call(
        _tg_kernel,
        out_shape=jax.ShapeDtypeStruct((B, 128), f32),
        grid_spec=pltpu.PrefetchScalarGridSpec(
            num_scalar_prefetch=0,
            grid=(B // Bb,),
            in_specs=[pl.BlockSpec((T, Bb, Dz), lambda i: (0, i, 0))]
            + [wspec(w) for w in weights],
            out_specs=pl.BlockSpec((Bb, 128), lambda i: (i, 0)),
            scratch_shapes=[
                pltpu.VMEM((T, Bb, 3 * H), f32),
                pltpu.VMEM((T, Bb, H), f32),
                pltpu.VMEM((Bb, T * H), f32),
            ],
        ),
        compiler_params=pltpu.CompilerParams(
            dimension_semantics=("parallel",)),
    )(z_tm, *weights)

    y = out[:, :T] + d_bl[0, 0]
    return y[:, :, None].astype(f32)
